# baseline (speedup 1.0000x reference)
import functools

import jax
import jax.numpy as jnp
import numpy as np
from jax import lax
from jax.experimental import pallas as pl
from jax.experimental.pallas import tpu as pltpu
from jax.experimental.pallas import tpu_sc as plsc

NUM_FIELDS = 26
FIELD_DIM = 100000
TOTAL_ROWS = NUM_FIELDS * FIELD_DIM
EMBED_DIM = 16
BATCH = 16384

NC = 2
NS = 16
NW = NC * NS
BW = BATCH // NW
CB = 64
NCHUNK = BATCH // CB
CPW = BW // CB
IPC = CB * NUM_FIELDS
IG = IPC // 128
GPC = CB // 16

_OFFSETS = np.array(
    (0, *np.cumsum([FIELD_DIM] * NUM_FIELDS)[:-1]), dtype=np.int32)


def _fm_body(idx_hbm, emb_hbm, fc_hbm, out_hbm,
             idx_v, rows_v, fc_v, out_v, sem_e, sem_f):
    wid = lax.axis_index("s") * NC + lax.axis_index("c")

    def chunk_body(c, _):
        gc = wid * CPW + c
        pltpu.sync_copy(idx_hbm.at[gc], idx_v)
        descs = []
        for g in range(IG):
            descs.append(pltpu.async_copy(
                emb_hbm.at[idx_v.at[g]],
                rows_v.at[pl.ds(g * 128, 128)], sem_e))
            descs.append(pltpu.async_copy(
                fc_hbm.at[idx_v.at[g]],
                fc_v.at[pl.ds(g * 128, 128)], sem_f))
        for d in descs:
            d.wait()

        lane = lax.iota(jnp.int32, 16)

        def group_body(g, _):
            o = g * 16
            lin = fc_v[pl.ds(o, 16)]
            for f in range(1, NUM_FIELDS):
                lin = lin + fc_v[pl.ds(f * CB + o, 16)]
            acc = jnp.zeros((16,), jnp.float32)
            for r in range(16):
                b = o + r
                e = rows_v[b]
                s = e
                q = e * e
                for f in range(1, NUM_FIELDS):
                    e = rows_v[f * CB + b]
                    s = s + e
                    q = q + e * e
                acc = jnp.where(lane == r, jnp.sum(s * s - q), acc)
            out_v[pl.ds(c * CB + o, 16)] = lin + 0.5 * acc
            return 0

        lax.fori_loop(0, GPC, group_body, 0)
        return 0

    lax.fori_loop(0, CPW, chunk_body, 0)
    pltpu.sync_copy(out_v, out_hbm.at[pl.ds(wid * BW, BW)])


@jax.jit
def _fm(idx_fm, emb_table, fc_flat):
    mesh = plsc.VectorSubcoreMesh(
        core_axis_name="c", subcore_axis_name="s",
        num_cores=NC, num_subcores=NS)
    f = functools.partial(
        pl.kernel,
        out_type=jax.ShapeDtypeStruct((BATCH,), jnp.float32),
        mesh=mesh,
        compiler_params=pltpu.CompilerParams(
            needs_layout_passes=False, use_tc_tiling_on_sc=False),
        scratch_types=[
            pltpu.VMEM((IG, 128), jnp.int32),
            pltpu.VMEM((IPC, EMBED_DIM), jnp.float32),
            pltpu.VMEM((IPC,), jnp.float32),
            pltpu.VMEM((BW,), jnp.float32),
            pltpu.SemaphoreType.DMA,
            pltpu.SemaphoreType.DMA,
        ],
    )(_fm_body)
    return f(idx_fm, emb_table, fc_flat)


def kernel(x, emb_table, fc_table, bias):
    idx = x.astype(jnp.int32) + jnp.asarray(_OFFSETS)[None, :]
    idx_fm = idx.reshape(NCHUNK, CB, NUM_FIELDS).transpose(0, 2, 1)
    idx_fm = idx_fm.reshape(NCHUNK, IG, 128)
    out = _fm(idx_fm, emb_table, fc_table[:, 0])
    return out[:, None] + bias[None, :]


# speedup vs baseline: 1.0074x; 1.0074x over previous
import functools

import jax
import jax.numpy as jnp
import numpy as np
from jax import lax
from jax.experimental import pallas as pl
from jax.experimental.pallas import tpu as pltpu
from jax.experimental.pallas import tpu_sc as plsc

NUM_FIELDS = 26
FIELD_DIM = 100000
TOTAL_ROWS = NUM_FIELDS * FIELD_DIM
EMBED_DIM = 16
BATCH = 16384

NC = 2
NS = 16
NW = NC * NS
BW = BATCH // NW
CB = 64
NCHUNK = BATCH // CB
CPW = BW // CB
IPC = CB * NUM_FIELDS
IG = IPC // 128
GPC = CB // 16
NBUF = 2

_OFFSETS = np.array(
    (0, *np.cumsum([FIELD_DIM] * NUM_FIELDS)[:-1]), dtype=np.int32)


def _fm_body(idx_hbm, emb_hbm, fc_hbm, out_hbm,
             idx_v, rows_v, fc_v, out_v, sem0, sem1):
    wid = lax.axis_index("s") * NC + lax.axis_index("c")
    pltpu.sync_copy(idx_hbm.at[wid], idx_v)
    sems = (sem0, sem1)

    def issue(c, b):
        for g in range(IG):
            pltpu.async_copy(
                emb_hbm.at[idx_v.at[c * IG + g]],
                rows_v.at[b, pl.ds(g * 128, 128)], sems[b])
            pltpu.async_copy(
                fc_hbm.at[idx_v.at[c * IG + g]],
                fc_v.at[b, pl.ds(g * 128, 128)], sems[b])

    def drain(b):
        pltpu.make_async_copy(
            emb_hbm.at[pl.ds(0, IPC)], rows_v.at[b], sems[b]).wait()
        pltpu.make_async_copy(
            fc_hbm.at[pl.ds(0, IPC)], fc_v.at[b], sems[b]).wait()

    def compute(c, b):
        lane = lax.iota(jnp.int32, 16)

        def group_body(g, _):
            o = g * 16
            lin = fc_v[b, pl.ds(o, 16)]
            for f in range(1, NUM_FIELDS):
                lin = lin + fc_v[b, pl.ds(f * CB + o, 16)]
            acc = jnp.zeros((16,), jnp.float32)
            for r in range(16):
                row = o + r
                e = rows_v[b, row]
                s = e
                q = e * e
                for f in range(1, NUM_FIELDS):
                    e = rows_v[b, f * CB + row]
                    s = s + e
                    q = q + e * e
                acc = jnp.where(lane == r, jnp.sum(s * s - q), acc)
            out_v[pl.ds(c * CB + o, 16)] = lin + 0.5 * acc
            return 0

        lax.fori_loop(0, GPC, group_body, 0)

    for b in range(NBUF):
        issue(b, b)

    def body(i, _):
        c = i * NBUF
        for b in range(NBUF):
            drain(b)
            compute(c + b, b)

            @pl.when(c + b + NBUF < CPW)
            def _():
                issue(c + b + NBUF, b)
        return 0

    lax.fori_loop(0, CPW // NBUF, body, 0)
    pltpu.sync_copy(out_v, out_hbm.at[pl.ds(wid * BW, BW)])


@jax.jit
def _fm(idx_fm, emb_table, fc_flat):
    mesh = plsc.VectorSubcoreMesh(
        core_axis_name="c", subcore_axis_name="s",
        num_cores=NC, num_subcores=NS)
    f = functools.partial(
        pl.kernel,
        out_type=jax.ShapeDtypeStruct((BATCH,), jnp.float32),
        mesh=mesh,
        compiler_params=pltpu.CompilerParams(
            needs_layout_passes=False, use_tc_tiling_on_sc=False),
        scratch_types=[
            pltpu.VMEM((CPW * IG, 128), jnp.int32),
            pltpu.VMEM((NBUF, IPC, EMBED_DIM), jnp.float32),
            pltpu.VMEM((NBUF, IPC), jnp.float32),
            pltpu.VMEM((BW,), jnp.float32),
            pltpu.SemaphoreType.DMA,
            pltpu.SemaphoreType.DMA,
        ],
    )(_fm_body)
    return f(idx_fm, emb_table, fc_flat)


def kernel(x, emb_table, fc_table, bias):
    idx = x.astype(jnp.int32) + jnp.asarray(_OFFSETS)[None, :]
    idx_fm = idx.reshape(NCHUNK, CB, NUM_FIELDS).transpose(0, 2, 1)
    idx_fm = idx_fm.reshape(NW, CPW * IG, 128)
    out = _fm(idx_fm, emb_table, fc_table[:, 0])
    return out[:, None] + bias[None, :]
